# baseline (speedup 1.0000x reference)
import dataclasses
import functools

import jax
import jax.numpy as jnp
from jax import lax
from jax.experimental import pallas as pl
from jax.experimental.pallas import tpu as pltpu
from jax.experimental.pallas import tpu_sc as plsc

B, L, K, C, H, W_ = 16, 1000, 17, 96, 64, 64
TOPK = 4
HIDDEN = 64
IN_DIM = K * (TOPK + 1)
N = B * L * K
NT = 4 * N
CHUNK = 64
NW = 32
CPW = 134
NCHUNK_PAD = NW * (CPW + 2)
N_PAD = NW * CPW * CHUNK
NT_PAD = 4 * NCHUNK_PAD * CHUNK
NG = 4 * CHUNK // 128
LANES = 16


def _sc_gather_combine(idx, w, table):
    mesh = plsc.VectorSubcoreMesh(core_axis_name="c", subcore_axis_name="s")
    cp = pltpu.CompilerParams()
    if "needs_layout_passes" in pltpu.CompilerParams.__dataclass_fields__:
        cp = dataclasses.replace(cp, needs_layout_passes=False)
    if "use_tc_tiling_on_sc" in pltpu.CompilerParams.__dataclass_fields__:
        cp = dataclasses.replace(cp, use_tc_tiling_on_sc=False)

    @functools.partial(
        pl.kernel,
        compiler_params=cp,
        out_type=jax.ShapeDtypeStruct((N_PAD, C), jnp.float32),
        mesh=mesh,
        scratch_types=(
            [pltpu.VMEM((4 * CHUNK,), jnp.int32)] * 2
            + [pltpu.VMEM((4 * CHUNK,), jnp.float32)] * 2
            + [pltpu.VMEM((4 * CHUNK, C), jnp.float32)] * 2
            + [pltpu.VMEM((CHUNK, C), jnp.float32)] * 2
            + [pltpu.SemaphoreType.DMA] * 6
        ),
    )
    def k(idx_hbm, w_hbm, table_hbm, sv_hbm,
          idx0, idx1, w0, w1, rows0, rows1, out0, out1,
          siw0, siw1, sg0, sg1, swb0, swb1):
        wid = lax.axis_index("c") * 16 + lax.axis_index("s")
        bufs = [(idx0, w0, rows0, out0, siw0, sg0, swb0),
                (idx1, w1, rows1, out1, siw1, sg1, swb1)]

        def start_idxw(b, cid):
            idx_v, w_v, _, _, siw, _, _ = bufs[b]
            t0 = cid * (4 * CHUNK)
            pltpu.make_async_copy(
                idx_hbm.at[pl.ds(t0, 4 * CHUNK)], idx_v, siw).start()
            pltpu.make_async_copy(
                w_hbm.at[pl.ds(t0, 4 * CHUNK)], w_v, siw).start()

        def wait_idxw(b):
            idx_v, w_v, _, _, siw, _, _ = bufs[b]
            pltpu.make_async_copy(
                idx_hbm.at[pl.ds(0, 4 * CHUNK)], idx_v, siw).wait()
            pltpu.make_async_copy(
                w_hbm.at[pl.ds(0, 4 * CHUNK)], w_v, siw).wait()

        def gather_copies(b):
            idx_v, _, rows_v, _, _, sg, _ = bufs[b]
            return [pltpu.make_async_copy(
                        table_hbm.at[idx_v.at[pl.ds(128 * g, 128)]],
                        rows_v.at[pl.ds(128 * g, 128)], sg)
                    for g in range(NG)]

        def start_wb(b, cid):
            _, _, _, out_v, _, _, swb = bufs[b]
            pltpu.make_async_copy(
                out_v, sv_hbm.at[pl.ds(cid * CHUNK, CHUNK)], swb).start()

        def wait_wb(b):
            _, _, _, out_v, _, _, swb = bufs[b]
            pltpu.make_async_copy(
                out_v, sv_hbm.at[pl.ds(0, CHUNK)], swb).wait()

        def compute(b):
            _, w_v, rows_v, out_v, _, _, _ = bufs[b]

            @pl.loop(0, CHUNK)
            def _(i):
                r0 = 4 * i
                wv = [plsc.load_gather(w_v, [jnp.full((LANES,), r0 + t,
                                                      jnp.int32)])
                      for t in range(4)]
                for c6 in range(C // LANES):
                    sl = pl.ds(LANES * c6, LANES)
                    acc = wv[0] * rows_v[r0, sl]
                    for t in range(1, 4):
                        acc = acc + wv[t] * rows_v[r0 + t, sl]
                    out_v[i, sl] = acc

        def phase(b, cid, first):
            ob = 1 - b
            wait_idxw(ob)
            for cp in gather_copies(ob):
                cp.start()
            for cp in gather_copies(b):
                cp.wait()
            if not first:
                wait_wb(b)
            compute(b)
            start_wb(b, cid)
            start_idxw(b, cid + 2 * NW)

        start_idxw(0, wid)
        start_idxw(1, wid + NW)
        wait_idxw(0)
        for cp in gather_copies(0):
            cp.start()
        phase(0, wid, True)
        phase(1, wid + NW, True)

        @pl.loop(1, CPW // 2)
        def _(u):
            c0 = wid + (2 * u) * NW
            phase(0, c0, False)
            phase(1, c0 + NW, False)

        for cp in gather_copies(0):
            cp.wait()
        wait_idxw(1)
        wait_wb(0)
        wait_wb(1)

    return k(idx, w, table)


ROWS = 64
GRID = B * L // ROWS


def _tc_head(sv_flat, scores2, W1, b1r, W2, b2r):
    def body(sv_ref, sc_ref, w1_ref, b1_ref, w2_ref, b2_ref, out_ref):
        v = sv_ref[...].reshape(ROWS, K, C)
        iot = lax.broadcasted_iota(jnp.int32, v.shape, 2)
        tops = []
        for _ in range(TOPK):
            m = jnp.max(v, axis=-1, keepdims=True)
            tops.append(m)
            amax = jnp.argmax(v, axis=-1)[..., None]
            v = jnp.where(iot == amax, -jnp.inf, v)
        mean = (tops[0] + tops[1] + tops[2] + tops[3]) * 0.25
        stat = jnp.concatenate(tops + [mean], axis=-1)
        x85 = stat.reshape(ROWS, IN_DIM)
        h = lax.dot_general(x85, w1_ref[...], (((1,), (1,)), ((), ())),
                            preferred_element_type=jnp.float32) + b1_ref[...]
        h = jnp.maximum(h, 0.0)
        q = jnp.sum(h * w2_ref[...], axis=-1, keepdims=True) + b2_ref[0]
        out_ref[...] = sc_ref[...] + q

    return pl.pallas_call(
        body,
        grid=(GRID,),
        in_specs=[
            pl.BlockSpec((ROWS * K, C), lambda i: (i, 0)),
            pl.BlockSpec((ROWS, 1), lambda i: (i, 0)),
            pl.BlockSpec((HIDDEN, IN_DIM), lambda i: (0, 0)),
            pl.BlockSpec((1, HIDDEN), lambda i: (0, 0)),
            pl.BlockSpec((1, HIDDEN), lambda i: (0, 0)),
            pl.BlockSpec(memory_space=pltpu.SMEM),
        ],
        out_specs=pl.BlockSpec((ROWS, 1), lambda i: (i, 0)),
        out_shape=jax.ShapeDtypeStruct((B * L, 1), jnp.float32),
    )(sv_flat, scores2, W1, b1r, W2, b2r)


def kernel(scores, pred_poses, feat, W1, b1, W2, b2):
    table = feat.transpose(0, 2, 3, 1).reshape(B * H * W_, C)
    pp = pred_poses.reshape(B, L, K, 2)
    ix = pp[..., 0] * W_ - 0.5
    iy = pp[..., 1] * H - 0.5
    ix0 = jnp.floor(ix)
    iy0 = jnp.floor(iy)
    wx1 = ix - ix0
    wy1 = iy - iy0
    boff = (jnp.arange(B, dtype=jnp.int32) * (H * W_))[:, None, None]
    idx_list, w_list = [], []
    for dy in (0, 1):
        for dx in (0, 1):
            xt = ix0 + dx
            yt = iy0 + dy
            valid = (xt >= 0) & (xt <= W_ - 1) & (yt >= 0) & (yt <= H - 1)
            xi = jnp.clip(xt, 0, W_ - 1).astype(jnp.int32)
            yi = jnp.clip(yt, 0, H - 1).astype(jnp.int32)
            wgt = (wx1 if dx else 1.0 - wx1) * (wy1 if dy else 1.0 - wy1)
            idx_list.append(boff + yi * W_ + xi)
            w_list.append(wgt * valid.astype(jnp.float32))
    idx = jnp.pad(jnp.stack(idx_list, axis=-1).reshape(NT), (0, NT_PAD - NT))
    w = jnp.pad(jnp.stack(w_list, axis=-1).reshape(NT), (0, NT_PAD - NT))

    sv = _sc_gather_combine(idx, w, table)
    out = _tc_head(sv, scores.reshape(B * L, 1),
                   W1, b1.reshape(1, HIDDEN), W2, b2)
    return out.reshape(B, L, 1)


# speedup vs baseline: 1.1476x; 1.1476x over previous
import dataclasses
import functools

import jax
import jax.numpy as jnp
from jax import lax
from jax.experimental import pallas as pl
from jax.experimental.pallas import tpu as pltpu
from jax.experimental.pallas import tpu_sc as plsc

B, L, K, C, H, W_ = 16, 1000, 17, 96, 64, 64
TOPK = 4
HIDDEN = 64
IN_DIM = K * (TOPK + 1)
N = B * L * K
NT = 4 * N
CHUNK = 64
NW = 32
CPW = 134
NCHUNK_PAD = NW * (CPW + 2)
N_PAD = NW * CPW * CHUNK
NT_PAD = 4 * NCHUNK_PAD * CHUNK
NG = 4 * CHUNK // 128
LANES = 16


def _sc_gather_combine(idx, w, table):
    mesh = plsc.VectorSubcoreMesh(core_axis_name="c", subcore_axis_name="s")
    cp = pltpu.CompilerParams()
    if "needs_layout_passes" in pltpu.CompilerParams.__dataclass_fields__:
        cp = dataclasses.replace(cp, needs_layout_passes=False)
    if "use_tc_tiling_on_sc" in pltpu.CompilerParams.__dataclass_fields__:
        cp = dataclasses.replace(cp, use_tc_tiling_on_sc=False)

    @functools.partial(
        pl.kernel,
        compiler_params=cp,
        out_type=jax.ShapeDtypeStruct((N_PAD, C), jnp.float32),
        mesh=mesh,
        scratch_types=(
            [pltpu.VMEM((4 * CHUNK,), jnp.int32)] * 2
            + [pltpu.VMEM((4 * CHUNK,), jnp.float32)] * 2
            + [pltpu.VMEM((4 * CHUNK, C), jnp.float32)] * 2
            + [pltpu.VMEM((CHUNK, C), jnp.float32)] * 2
            + [pltpu.SemaphoreType.DMA] * 6
        ),
    )
    def k(idx_hbm, w_hbm, table_hbm, sv_hbm,
          idx0, idx1, w0, w1, rows0, rows1, out0, out1,
          siw0, siw1, sg0, sg1, swb0, swb1):
        wid = lax.axis_index("c") * 16 + lax.axis_index("s")
        bufs = [(idx0, w0, rows0, out0, siw0, sg0, swb0),
                (idx1, w1, rows1, out1, siw1, sg1, swb1)]

        def start_idxw(b, cid):
            idx_v, w_v, _, _, siw, _, _ = bufs[b]
            t0 = cid * (4 * CHUNK)
            pltpu.make_async_copy(
                idx_hbm.at[pl.ds(t0, 4 * CHUNK)], idx_v, siw).start()
            pltpu.make_async_copy(
                w_hbm.at[pl.ds(t0, 4 * CHUNK)], w_v, siw).start()

        def wait_idxw(b):
            idx_v, w_v, _, _, siw, _, _ = bufs[b]
            pltpu.make_async_copy(
                idx_hbm.at[pl.ds(0, 4 * CHUNK)], idx_v, siw).wait()
            pltpu.make_async_copy(
                w_hbm.at[pl.ds(0, 4 * CHUNK)], w_v, siw).wait()

        def gather_copies(b):
            idx_v, _, rows_v, _, _, sg, _ = bufs[b]
            return [pltpu.make_async_copy(
                        table_hbm.at[idx_v.at[pl.ds(128 * g, 128)]],
                        rows_v.at[pl.ds(128 * g, 128)], sg)
                    for g in range(NG)]

        def start_wb(b, cid):
            _, _, _, out_v, _, _, swb = bufs[b]
            pltpu.make_async_copy(
                out_v, sv_hbm.at[pl.ds(cid * CHUNK, CHUNK)], swb).start()

        def wait_wb(b):
            _, _, _, out_v, _, _, swb = bufs[b]
            pltpu.make_async_copy(
                out_v, sv_hbm.at[pl.ds(0, CHUNK)], swb).wait()

        def compute(b):
            _, w_v, rows_v, out_v, _, _, _ = bufs[b]

            @plsc.parallel_loop(0, CHUNK, unroll=4)
            def _(i):
                r0 = 4 * i
                wv = [plsc.load_gather(w_v, [jnp.full((LANES,), r0 + t,
                                                      jnp.int32)])
                      for t in range(4)]
                for c6 in range(C // LANES):
                    sl = pl.ds(LANES * c6, LANES)
                    acc = wv[0] * rows_v[r0, sl]
                    for t in range(1, 4):
                        acc = acc + wv[t] * rows_v[r0 + t, sl]
                    out_v[i, sl] = acc

        def phase(b, cid, first):
            ob = 1 - b
            wait_idxw(ob)
            for cp in gather_copies(ob):
                cp.start()
            for cp in gather_copies(b):
                cp.wait()
            if not first:
                wait_wb(b)
            compute(b)
            start_wb(b, cid)
            start_idxw(b, cid + 2 * NW)

        start_idxw(0, wid)
        start_idxw(1, wid + NW)
        wait_idxw(0)
        for cp in gather_copies(0):
            cp.start()
        phase(0, wid, True)
        phase(1, wid + NW, True)

        @pl.loop(1, CPW // 2)
        def _(u):
            c0 = wid + (2 * u) * NW
            phase(0, c0, False)
            phase(1, c0 + NW, False)

        for cp in gather_copies(0):
            cp.wait()
        wait_idxw(1)
        wait_wb(0)
        wait_wb(1)

    return k(idx, w, table)


ROWS = 64
GRID = B * L // ROWS


def _tc_head(sv_flat, scores2, W1, b1r, W2, b2r):
    def body(sv_ref, sc_ref, w1_ref, b1_ref, w2_ref, b2_ref, out_ref):
        v = sv_ref[...].reshape(ROWS, K, C)
        iot = lax.broadcasted_iota(jnp.int32, v.shape, 2)
        tops = []
        for _ in range(TOPK):
            m = jnp.max(v, axis=-1, keepdims=True)
            tops.append(m)
            amax = jnp.argmax(v, axis=-1)[..., None]
            v = jnp.where(iot == amax, -jnp.inf, v)
        mean = (tops[0] + tops[1] + tops[2] + tops[3]) * 0.25
        stat = jnp.concatenate(tops + [mean], axis=-1)
        x85 = stat.reshape(ROWS, IN_DIM)
        h = lax.dot_general(x85, w1_ref[...], (((1,), (1,)), ((), ())),
                            preferred_element_type=jnp.float32) + b1_ref[...]
        h = jnp.maximum(h, 0.0)
        q = jnp.sum(h * w2_ref[...], axis=-1, keepdims=True) + b2_ref[0]
        out_ref[...] = sc_ref[...] + q

    return pl.pallas_call(
        body,
        grid=(GRID,),
        in_specs=[
            pl.BlockSpec((ROWS * K, C), lambda i: (i, 0)),
            pl.BlockSpec((ROWS, 1), lambda i: (i, 0)),
            pl.BlockSpec((HIDDEN, IN_DIM), lambda i: (0, 0)),
            pl.BlockSpec((1, HIDDEN), lambda i: (0, 0)),
            pl.BlockSpec((1, HIDDEN), lambda i: (0, 0)),
            pl.BlockSpec(memory_space=pltpu.SMEM),
        ],
        out_specs=pl.BlockSpec((ROWS, 1), lambda i: (i, 0)),
        out_shape=jax.ShapeDtypeStruct((B * L, 1), jnp.float32),
    )(sv_flat, scores2, W1, b1r, W2, b2r)


def kernel(scores, pred_poses, feat, W1, b1, W2, b2):
    table = feat.transpose(0, 2, 3, 1).reshape(B * H * W_, C)
    pp = pred_poses.reshape(B, L, K, 2)
    ix = pp[..., 0] * W_ - 0.5
    iy = pp[..., 1] * H - 0.5
    ix0 = jnp.floor(ix)
    iy0 = jnp.floor(iy)
    wx1 = ix - ix0
    wy1 = iy - iy0
    boff = (jnp.arange(B, dtype=jnp.int32) * (H * W_))[:, None, None]
    idx_list, w_list = [], []
    for dy in (0, 1):
        for dx in (0, 1):
            xt = ix0 + dx
            yt = iy0 + dy
            valid = (xt >= 0) & (xt <= W_ - 1) & (yt >= 0) & (yt <= H - 1)
            xi = jnp.clip(xt, 0, W_ - 1).astype(jnp.int32)
            yi = jnp.clip(yt, 0, H - 1).astype(jnp.int32)
            wgt = (wx1 if dx else 1.0 - wx1) * (wy1 if dy else 1.0 - wy1)
            idx_list.append(boff + yi * W_ + xi)
            w_list.append(wgt * valid.astype(jnp.float32))
    idx = jnp.pad(jnp.stack(idx_list, axis=-1).reshape(NT), (0, NT_PAD - NT))
    w = jnp.pad(jnp.stack(w_list, axis=-1).reshape(NT), (0, NT_PAD - NT))

    sv = _sc_gather_combine(idx, w, table)
    out = _tc_head(sv, scores.reshape(B * L, 1),
                   W1, b1.reshape(1, HIDDEN), W2, b2)
    return out.reshape(B, L, 1)


# speedup vs baseline: 1.1480x; 1.0004x over previous
import dataclasses
import functools

import jax
import jax.numpy as jnp
from jax import lax
from jax.experimental import pallas as pl
from jax.experimental.pallas import tpu as pltpu
from jax.experimental.pallas import tpu_sc as plsc

B, L, K, C, H, W_ = 16, 1000, 17, 96, 64, 64
TOPK = 4
HIDDEN = 64
IN_DIM = K * (TOPK + 1)
N = B * L * K
NT = 4 * N
CHUNK = 64
NW = 32
CPW = 134
NCHUNK_PAD = NW * (CPW + 2)
N_PAD = NW * CPW * CHUNK
NT_PAD = 4 * NCHUNK_PAD * CHUNK
NG = 4 * CHUNK // 128
LANES = 16


def _sc_gather_combine(idx, w, table):
    mesh = plsc.VectorSubcoreMesh(core_axis_name="c", subcore_axis_name="s")
    cp = pltpu.CompilerParams()
    if "needs_layout_passes" in pltpu.CompilerParams.__dataclass_fields__:
        cp = dataclasses.replace(cp, needs_layout_passes=False)
    if "use_tc_tiling_on_sc" in pltpu.CompilerParams.__dataclass_fields__:
        cp = dataclasses.replace(cp, use_tc_tiling_on_sc=False)

    @functools.partial(
        pl.kernel,
        compiler_params=cp,
        out_type=jax.ShapeDtypeStruct((N_PAD, C), jnp.float32),
        mesh=mesh,
        scratch_types=(
            [pltpu.VMEM((4 * CHUNK,), jnp.int32)] * 2
            + [pltpu.VMEM((4 * CHUNK,), jnp.float32)] * 2
            + [pltpu.VMEM((4 * CHUNK, C), jnp.float32)] * 2
            + [pltpu.VMEM((CHUNK, C), jnp.float32)] * 2
            + [pltpu.SemaphoreType.DMA] * 6
        ),
    )
    def k(idx_hbm, w_hbm, table_hbm, sv_hbm,
          idx0, idx1, w0, w1, rows0, rows1, out0, out1,
          siw0, siw1, sg0, sg1, swb0, swb1):
        wid = lax.axis_index("c") * 16 + lax.axis_index("s")
        bufs = [(idx0, w0, rows0, out0, siw0, sg0, swb0),
                (idx1, w1, rows1, out1, siw1, sg1, swb1)]

        def start_idxw(b, cid):
            idx_v, w_v, _, _, siw, _, _ = bufs[b]
            t0 = cid * (4 * CHUNK)
            pltpu.make_async_copy(
                idx_hbm.at[pl.ds(t0, 4 * CHUNK)], idx_v, siw).start()
            pltpu.make_async_copy(
                w_hbm.at[pl.ds(t0, 4 * CHUNK)], w_v, siw).start()

        def wait_idxw(b):
            idx_v, w_v, _, _, siw, _, _ = bufs[b]
            pltpu.make_async_copy(
                idx_hbm.at[pl.ds(0, 4 * CHUNK)], idx_v, siw).wait()
            pltpu.make_async_copy(
                w_hbm.at[pl.ds(0, 4 * CHUNK)], w_v, siw).wait()

        def gather_copies(b):
            idx_v, _, rows_v, _, _, sg, _ = bufs[b]
            return [pltpu.make_async_copy(
                        table_hbm.at[idx_v], rows_v, sg)]

        def start_wb(b, cid):
            _, _, _, out_v, _, _, swb = bufs[b]
            pltpu.make_async_copy(
                out_v, sv_hbm.at[pl.ds(cid * CHUNK, CHUNK)], swb).start()

        def wait_wb(b):
            _, _, _, out_v, _, _, swb = bufs[b]
            pltpu.make_async_copy(
                out_v, sv_hbm.at[pl.ds(0, CHUNK)], swb).wait()

        def compute(b):
            _, w_v, rows_v, out_v, _, _, _ = bufs[b]

            @plsc.parallel_loop(0, CHUNK, unroll=4)
            def _(i):
                r0 = 4 * i
                wv = [plsc.load_gather(w_v, [jnp.full((LANES,), r0 + t,
                                                      jnp.int32)])
                      for t in range(4)]
                for c6 in range(C // LANES):
                    sl = pl.ds(LANES * c6, LANES)
                    acc = wv[0] * rows_v[r0, sl]
                    for t in range(1, 4):
                        acc = acc + wv[t] * rows_v[r0 + t, sl]
                    out_v[i, sl] = acc

        def phase(b, cid, first):
            ob = 1 - b
            wait_idxw(ob)
            for cp in gather_copies(ob):
                cp.start()
            for cp in gather_copies(b):
                cp.wait()
            if not first:
                wait_wb(b)
            compute(b)
            start_wb(b, cid)
            start_idxw(b, cid + 2 * NW)

        start_idxw(0, wid)
        start_idxw(1, wid + NW)
        wait_idxw(0)
        for cp in gather_copies(0):
            cp.start()
        phase(0, wid, True)
        phase(1, wid + NW, True)

        @pl.loop(1, CPW // 2)
        def _(u):
            c0 = wid + (2 * u) * NW
            phase(0, c0, False)
            phase(1, c0 + NW, False)

        for cp in gather_copies(0):
            cp.wait()
        wait_idxw(1)
        wait_wb(0)
        wait_wb(1)

    return k(idx, w, table)


ROWS = 64
GRID = B * L // ROWS


def _tc_head(sv_flat, scores2, W1, b1r, W2, b2r):
    def body(sv_ref, sc_ref, w1_ref, b1_ref, w2_ref, b2_ref, out_ref):
        v = sv_ref[...].reshape(ROWS, K, C)
        iot = lax.broadcasted_iota(jnp.int32, v.shape, 2)
        tops = []
        for _ in range(TOPK):
            m = jnp.max(v, axis=-1, keepdims=True)
            tops.append(m)
            amax = jnp.argmax(v, axis=-1)[..., None]
            v = jnp.where(iot == amax, -jnp.inf, v)
        mean = (tops[0] + tops[1] + tops[2] + tops[3]) * 0.25
        stat = jnp.concatenate(tops + [mean], axis=-1)
        x85 = stat.reshape(ROWS, IN_DIM)
        h = lax.dot_general(x85, w1_ref[...], (((1,), (1,)), ((), ())),
                            preferred_element_type=jnp.float32) + b1_ref[...]
        h = jnp.maximum(h, 0.0)
        q = jnp.sum(h * w2_ref[...], axis=-1, keepdims=True) + b2_ref[0]
        out_ref[...] = sc_ref[...] + q

    return pl.pallas_call(
        body,
        grid=(GRID,),
        in_specs=[
            pl.BlockSpec((ROWS * K, C), lambda i: (i, 0)),
            pl.BlockSpec((ROWS, 1), lambda i: (i, 0)),
            pl.BlockSpec((HIDDEN, IN_DIM), lambda i: (0, 0)),
            pl.BlockSpec((1, HIDDEN), lambda i: (0, 0)),
            pl.BlockSpec((1, HIDDEN), lambda i: (0, 0)),
            pl.BlockSpec(memory_space=pltpu.SMEM),
        ],
        out_specs=pl.BlockSpec((ROWS, 1), lambda i: (i, 0)),
        out_shape=jax.ShapeDtypeStruct((B * L, 1), jnp.float32),
    )(sv_flat, scores2, W1, b1r, W2, b2r)


def kernel(scores, pred_poses, feat, W1, b1, W2, b2):
    table = feat.transpose(0, 2, 3, 1).reshape(B * H * W_, C)
    pp = pred_poses.reshape(B, L, K, 2)
    ix = pp[..., 0] * W_ - 0.5
    iy = pp[..., 1] * H - 0.5
    ix0 = jnp.floor(ix)
    iy0 = jnp.floor(iy)
    wx1 = ix - ix0
    wy1 = iy - iy0
    boff = (jnp.arange(B, dtype=jnp.int32) * (H * W_))[:, None, None]
    idx_list, w_list = [], []
    for dy in (0, 1):
        for dx in (0, 1):
            xt = ix0 + dx
            yt = iy0 + dy
            valid = (xt >= 0) & (xt <= W_ - 1) & (yt >= 0) & (yt <= H - 1)
            xi = jnp.clip(xt, 0, W_ - 1).astype(jnp.int32)
            yi = jnp.clip(yt, 0, H - 1).astype(jnp.int32)
            wgt = (wx1 if dx else 1.0 - wx1) * (wy1 if dy else 1.0 - wy1)
            idx_list.append(boff + yi * W_ + xi)
            w_list.append(wgt * valid.astype(jnp.float32))
    idx = jnp.pad(jnp.stack(idx_list, axis=-1).reshape(NT), (0, NT_PAD - NT))
    w = jnp.pad(jnp.stack(w_list, axis=-1).reshape(NT), (0, NT_PAD - NT))

    sv = _sc_gather_combine(idx, w, table)
    out = _tc_head(sv, scores.reshape(B * L, 1),
                   W1, b1.reshape(1, HIDDEN), W2, b2)
    return out.reshape(B, L, 1)


# speedup vs baseline: 1.1545x; 1.0057x over previous
import dataclasses
import functools

import jax
import jax.numpy as jnp
from jax import lax
from jax.experimental import pallas as pl
from jax.experimental.pallas import tpu as pltpu
from jax.experimental.pallas import tpu_sc as plsc

B, L, K, C, H, W_ = 16, 1000, 17, 96, 64, 64
TOPK = 4
HIDDEN = 64
IN_DIM = K * (TOPK + 1)
N = B * L * K
NT = 4 * N
NI = 2 * N
CHUNK = 64
NW = 32
CPW = 134
NCHUNK_PAD = NW * (CPW + 2)
N_PAD = NW * CPW * CHUNK
NT_PAD = 4 * NCHUNK_PAD * CHUNK
NI_PAD = 2 * NCHUNK_PAD * CHUNK
LANES = 16


def _sc_gather_combine(idx, w, table):
    mesh = plsc.VectorSubcoreMesh(core_axis_name="c", subcore_axis_name="s")
    cp = pltpu.CompilerParams()
    if "needs_layout_passes" in pltpu.CompilerParams.__dataclass_fields__:
        cp = dataclasses.replace(cp, needs_layout_passes=False)
    if "use_tc_tiling_on_sc" in pltpu.CompilerParams.__dataclass_fields__:
        cp = dataclasses.replace(cp, use_tc_tiling_on_sc=False)

    @functools.partial(
        pl.kernel,
        compiler_params=cp,
        out_type=jax.ShapeDtypeStruct((N_PAD, C), jnp.float32),
        mesh=mesh,
        scratch_types=(
            [pltpu.VMEM((2 * CHUNK,), jnp.int32)] * 2
            + [pltpu.VMEM((4 * CHUNK,), jnp.float32)] * 2
            + [pltpu.VMEM((2 * CHUNK, 2 * C), jnp.float32)] * 2
            + [pltpu.VMEM((CHUNK, C), jnp.float32)] * 2
            + [pltpu.SemaphoreType.DMA] * 6
        ),
    )
    def k(idx_hbm, w_hbm, table_hbm, sv_hbm,
          idx0, idx1, w0, w1, rows0, rows1, out0, out1,
          siw0, siw1, sg0, sg1, swb0, swb1):
        wid = lax.axis_index("c") * 16 + lax.axis_index("s")
        bufs = [(idx0, w0, rows0, out0, siw0, sg0, swb0),
                (idx1, w1, rows1, out1, siw1, sg1, swb1)]

        def start_idxw(b, cid):
            idx_v, w_v, _, _, siw, _, _ = bufs[b]
            pltpu.make_async_copy(
                idx_hbm.at[pl.ds(cid * (2 * CHUNK), 2 * CHUNK)],
                idx_v, siw).start()
            pltpu.make_async_copy(
                w_hbm.at[pl.ds(cid * (4 * CHUNK), 4 * CHUNK)],
                w_v, siw).start()

        def wait_idxw(b):
            idx_v, w_v, _, _, siw, _, _ = bufs[b]
            pltpu.make_async_copy(
                idx_hbm.at[pl.ds(0, 2 * CHUNK)], idx_v, siw).wait()
            pltpu.make_async_copy(
                w_hbm.at[pl.ds(0, 4 * CHUNK)], w_v, siw).wait()

        def gather_copies(b):
            idx_v, _, rows_v, _, _, sg, _ = bufs[b]
            return [pltpu.make_async_copy(
                        table_hbm.at[idx_v], rows_v, sg)]

        def start_wb(b, cid):
            _, _, _, out_v, _, _, swb = bufs[b]
            pltpu.make_async_copy(
                out_v, sv_hbm.at[pl.ds(cid * CHUNK, CHUNK)], swb).start()

        def wait_wb(b):
            _, _, _, out_v, _, _, swb = bufs[b]
            pltpu.make_async_copy(
                out_v, sv_hbm.at[pl.ds(0, CHUNK)], swb).wait()

        def compute(b):
            _, w_v, rows_v, out_v, _, _, _ = bufs[b]

            @plsc.parallel_loop(0, CHUNK, unroll=4)
            def _(i):
                r0 = 2 * i
                wv = [plsc.load_gather(w_v, [jnp.full((LANES,), 4 * i + t,
                                                      jnp.int32)])
                      for t in range(4)]
                for c6 in range(C // LANES):
                    lo = pl.ds(LANES * c6, LANES)
                    hi = pl.ds(C + LANES * c6, LANES)
                    acc = (wv[0] * rows_v[r0, lo] + wv[1] * rows_v[r0, hi]
                           + wv[2] * rows_v[r0 + 1, lo]
                           + wv[3] * rows_v[r0 + 1, hi])
                    out_v[i, lo] = acc

        def phase(b, cid, first):
            ob = 1 - b
            wait_idxw(ob)
            for cp in gather_copies(ob):
                cp.start()
            for cp in gather_copies(b):
                cp.wait()
            if not first:
                wait_wb(b)
            compute(b)
            start_wb(b, cid)
            start_idxw(b, cid + 2 * NW)

        start_idxw(0, wid)
        start_idxw(1, wid + NW)
        wait_idxw(0)
        for cp in gather_copies(0):
            cp.start()
        phase(0, wid, True)
        phase(1, wid + NW, True)

        @pl.loop(1, CPW // 2)
        def _(u):
            c0 = wid + (2 * u) * NW
            phase(0, c0, False)
            phase(1, c0 + NW, False)

        for cp in gather_copies(0):
            cp.wait()
        wait_idxw(1)
        wait_wb(0)
        wait_wb(1)

    return k(idx, w, table)


ROWS = 64
GRID = B * L // ROWS


def _tc_head(sv_flat, scores2, W1, b1r, W2, b2r):
    def body(sv_ref, sc_ref, w1_ref, b1_ref, w2_ref, b2_ref, out_ref):
        v = sv_ref[...].reshape(ROWS, K, C)
        iot = lax.broadcasted_iota(jnp.int32, v.shape, 2)
        tops = []
        for _ in range(TOPK):
            m = jnp.max(v, axis=-1, keepdims=True)
            tops.append(m)
            amax = jnp.argmax(v, axis=-1)[..., None]
            v = jnp.where(iot == amax, -jnp.inf, v)
        mean = (tops[0] + tops[1] + tops[2] + tops[3]) * 0.25
        stat = jnp.concatenate(tops + [mean], axis=-1)
        x85 = stat.reshape(ROWS, IN_DIM)
        h = lax.dot_general(x85, w1_ref[...], (((1,), (1,)), ((), ())),
                            preferred_element_type=jnp.float32) + b1_ref[...]
        h = jnp.maximum(h, 0.0)
        q = jnp.sum(h * w2_ref[...], axis=-1, keepdims=True) + b2_ref[0]
        out_ref[...] = sc_ref[...] + q

    return pl.pallas_call(
        body,
        grid=(GRID,),
        in_specs=[
            pl.BlockSpec((ROWS * K, C), lambda i: (i, 0)),
            pl.BlockSpec((ROWS, 1), lambda i: (i, 0)),
            pl.BlockSpec((HIDDEN, IN_DIM), lambda i: (0, 0)),
            pl.BlockSpec((1, HIDDEN), lambda i: (0, 0)),
            pl.BlockSpec((1, HIDDEN), lambda i: (0, 0)),
            pl.BlockSpec(memory_space=pltpu.SMEM),
        ],
        out_specs=pl.BlockSpec((ROWS, 1), lambda i: (i, 0)),
        out_shape=jax.ShapeDtypeStruct((B * L, 1), jnp.float32),
    )(sv_flat, scores2, W1, b1r, W2, b2r)


def kernel(scores, pred_poses, feat, W1, b1, W2, b2):
    table = feat.transpose(0, 2, 3, 1).reshape(B * H * W_, C)
    table2 = jnp.concatenate([table[:-1], table[1:]], axis=1)
    pp = pred_poses.reshape(B, L, K, 2)
    ix = pp[..., 0] * W_ - 0.5
    iy = pp[..., 1] * H - 0.5
    ix0 = jnp.floor(ix)
    iy0 = jnp.floor(iy)
    wx1 = ix - ix0
    wy1 = iy - iy0
    boff = (jnp.arange(B, dtype=jnp.int32) * (H * W_))[:, None, None]

    xlo = jnp.clip(ix0, 0, W_ - 2).astype(jnp.int32)
    wl = (1.0 - wx1) * ((ix0 >= 0) & (ix0 <= W_ - 1)).astype(jnp.float32)
    wh = wx1 * ((ix0 + 1 >= 0) & (ix0 + 1 <= W_ - 1)).astype(jnp.float32)
    s0 = jnp.clip(ix0, 0, W_ - 1).astype(jnp.int32) - xlo
    s1 = jnp.clip(ix0 + 1, 0, W_ - 1).astype(jnp.int32) - xlo
    wslot0 = wl * (s0 == 0).astype(jnp.float32) \
        + wh * (s1 == 0).astype(jnp.float32)
    wslot1 = wl * (s0 == 1).astype(jnp.float32) \
        + wh * (s1 == 1).astype(jnp.float32)

    idx_list, w_list = [], []
    for dy in (0, 1):
        yt = iy0 + dy
        vy = ((yt >= 0) & (yt <= H - 1)).astype(jnp.float32)
        yi = jnp.clip(yt, 0, H - 1).astype(jnp.int32)
        wy = (wy1 if dy else 1.0 - wy1) * vy
        idx_list.append(boff + yi * W_ + xlo)
        w_list.append(wy * wslot0)
        w_list.append(wy * wslot1)
    idx = jnp.pad(jnp.stack(idx_list, axis=-1).reshape(NI), (0, NI_PAD - NI))
    w = jnp.pad(jnp.stack(w_list, axis=-1).reshape(NT), (0, NT_PAD - NT))

    sv = _sc_gather_combine(idx, w, table2)
    out = _tc_head(sv, scores.reshape(B * L, 1),
                   W1, b1.reshape(1, HIDDEN), W2, b2)
    return out.reshape(B, L, 1)
